# baseline (speedup 1.0000x reference)
import jax
import jax.numpy as jnp
from jax.experimental import pallas as pl
from jax.experimental.pallas import tpu as pltpu

B, D, R = 512, 512, 256
TOP_K_FACTS, TOP_K_RULES, FACT_TEMP = 2, 8, 0.7
RC = 8


def _lane_prod(t):
    n = t.shape[-1]
    while n > 1:
        h = n // 2
        t = t[..., :h] * t[..., h:n]
        n = h
    return t[..., 0]


def _body(facts_ref, fl_ref, agg_ref, rs_ref, w_ref, gamma_ref, beta_ref,
          out_ref, mask_ref, andt_ref, ort_ref):
    f = facts_ref[...]
    fl = fl_ref[...]

    z = fl * (1.0 / FACT_TEMP)
    z = z - jnp.max(z, axis=1, keepdims=True)
    e = jnp.exp(z)
    p = e / jnp.sum(e, axis=1, keepdims=True)
    mask = jnp.minimum(TOP_K_FACTS * p, 1.0)
    mask_ref[...] = mask
    denom = jnp.sum(mask, axis=1, keepdims=True) + 1e-8

    kofnt = jax.lax.dot_general(
        mask, f, (((1,), (1,)), ((), ())),
        precision=jax.lax.Precision.HIGHEST,
        preferred_element_type=jnp.float32) / denom

    aw = agg_ref[...]
    aw = aw - jnp.max(aw, axis=1, keepdims=True)
    ea = jnp.exp(aw)
    w = ea / jnp.sum(ea, axis=1, keepdims=True)
    strength = jax.nn.sigmoid(rs_ref[...])

    def chunk(i, carry):
        mc = mask_ref[pl.ds(i * RC, RC), :]
        sel = f[:, None, :] * mc[None, :, :]
        and_c = _lane_prod(sel + (1.0 - mc)[None, :, :])
        or_c = 1.0 - _lane_prod((1.0 - sel) + 1e-8)
        andt_ref[pl.ds(i * RC, RC), :] = and_c.T
        ort_ref[pl.ds(i * RC, RC), :] = or_c.T
        return carry

    jax.lax.fori_loop(0, R // RC, chunk, 0, unroll=False)

    mixedt = (andt_ref[...] * w[:, 0:1] + ort_ref[...] * w[:, 1:2]
              + kofnt * w[:, 2:3]) * strength
    act = mixedt.T

    iota = jax.lax.broadcasted_iota(jnp.int32, (B, R), 1)
    removed = jnp.zeros((B, R), jnp.bool_)
    for _ in range(TOP_K_RULES):
        cur = jnp.where(removed, -jnp.inf, act)
        m = jnp.max(cur, axis=1, keepdims=True)
        cand = jnp.where(cur == m, iota, R)
        sel_idx = jnp.min(cand, axis=1, keepdims=True)
        removed = removed | (iota == sel_idx)
    gated = jnp.where(removed, act, 0.0)

    proj = jax.lax.dot_general(
        f, w_ref[...], (((1,), (1,)), ((), ())),
        precision=jax.lax.Precision.HIGHEST,
        preferred_element_type=jnp.float32)
    pre = proj + gated
    mu = jnp.mean(pre, axis=1, keepdims=True)
    cen = pre - mu
    var = jnp.mean(cen * cen, axis=1, keepdims=True)
    out_ref[...] = cen * jax.lax.rsqrt(var + 1e-5) * gamma_ref[...] \
        + beta_ref[...]


@jax.jit
def kernel(facts, fact_logits, aggregator_logits, rule_strength_raw, W_proj,
           ln_gamma, ln_beta):
    rs = rule_strength_raw.reshape(R, 1)
    gamma = ln_gamma.reshape(1, R)
    beta = ln_beta.reshape(1, R)
    return pl.pallas_call(
        _body,
        out_shape=jax.ShapeDtypeStruct((B, R), jnp.float32),
        scratch_shapes=[
            pltpu.VMEM((R, D), jnp.float32),
            pltpu.VMEM((R, B), jnp.float32),
            pltpu.VMEM((R, B), jnp.float32),
        ],
    )(facts, fact_logits, aggregator_logits, rs, W_proj, gamma, beta)


# speedup vs baseline: 2.8461x; 2.8461x over previous
import jax
import jax.numpy as jnp
from jax.experimental import pallas as pl
from jax.experimental.pallas import tpu as pltpu

B, D, R = 512, 512, 256
TOP_K_FACTS, TOP_K_RULES, FACT_TEMP = 2, 8, 0.7


def _sub_prod(t):
    n = t.shape[0]
    while n > 1:
        h = n // 2
        t = t[:h, :] * t[h:n, :]
        n = h
    return t


def _dot_t(a, b):
    return jax.lax.dot_general(
        a, b, (((1,), (1,)), ((), ())),
        precision=jax.lax.Precision.HIGHEST,
        preferred_element_type=jnp.float32)


def _body(facts_ref, fl_ref, agg_ref, rs_ref, w_ref, gamma_ref, beta_ref,
          out_ref, mask_ref, andt_ref, ort_ref):
    f = facts_ref[...]
    fl = fl_ref[...]

    z = fl * (1.0 / FACT_TEMP)
    z = z - jnp.max(z, axis=1, keepdims=True)
    e = jnp.exp(z)
    p = e / jnp.sum(e, axis=1, keepdims=True)
    mask = jnp.minimum(TOP_K_FACTS * p, 1.0)
    mask_ref[...] = mask
    denom = jnp.sum(mask, axis=1, keepdims=True) + 1e-8

    kofnt = _dot_t(mask, f) / denom

    ft = f.T

    def rule(r, carry):
        m_col = mask_ref[pl.ds(r, 1), :].T
        sel = ft * m_col
        and_t = sel + (1.0 - m_col)
        or_t = (1.0 - sel) + 1e-8
        andt_ref[pl.ds(r, 1), :] = _sub_prod(and_t)
        ort_ref[pl.ds(r, 1), :] = _sub_prod(or_t)
        return carry

    jax.lax.fori_loop(0, R, rule, 0, unroll=False)

    aw = agg_ref[...]
    aw = aw - jnp.max(aw, axis=1, keepdims=True)
    ea = jnp.exp(aw)
    w = ea / jnp.sum(ea, axis=1, keepdims=True)
    strength = jax.nn.sigmoid(rs_ref[...])

    mixedt = (andt_ref[...] * w[:, 0:1]
              + (1.0 - ort_ref[...]) * w[:, 1:2]
              + kofnt * w[:, 2:3]) * strength
    act = mixedt.T

    iota = jax.lax.broadcasted_iota(jnp.int32, (B, R), 1)
    removed = jnp.zeros((B, R), jnp.bool_)
    for _ in range(TOP_K_RULES):
        cur = jnp.where(removed, -jnp.inf, act)
        m = jnp.max(cur, axis=1, keepdims=True)
        cand = jnp.where(cur == m, iota, R)
        sel_idx = jnp.min(cand, axis=1, keepdims=True)
        removed = removed | (iota == sel_idx)
    gated = jnp.where(removed, act, 0.0)

    proj = _dot_t(f, w_ref[...])
    pre = proj + gated
    mu = jnp.mean(pre, axis=1, keepdims=True)
    cen = pre - mu
    var = jnp.mean(cen * cen, axis=1, keepdims=True)
    out_ref[...] = cen * jax.lax.rsqrt(var + 1e-5) * gamma_ref[...] \
        + beta_ref[...]


@jax.jit
def kernel(facts, fact_logits, aggregator_logits, rule_strength_raw, W_proj,
           ln_gamma, ln_beta):
    rs = rule_strength_raw.reshape(R, 1)
    gamma = ln_gamma.reshape(1, R)
    beta = ln_beta.reshape(1, R)
    return pl.pallas_call(
        _body,
        out_shape=jax.ShapeDtypeStruct((B, R), jnp.float32),
        scratch_shapes=[
            pltpu.VMEM((R, D), jnp.float32),
            pltpu.VMEM((R, B), jnp.float32),
            pltpu.VMEM((R, B), jnp.float32),
        ],
    )(facts, fact_logits, aggregator_logits, rs, W_proj, gamma, beta)


# speedup vs baseline: 3.3394x; 1.1733x over previous
import jax
import jax.numpy as jnp
from jax.experimental import pallas as pl
from jax.experimental.pallas import tpu as pltpu

B, D, R = 512, 512, 256
TOP_K_FACTS, TOP_K_RULES, FACT_TEMP = 2, 8, 0.7


def _sub_prod(t):
    n = t.shape[0]
    while n > 1:
        h = n // 2
        t = t[:h, :] * t[h:n, :]
        n = h
    return t


def _dot_t(a, b):
    return jax.lax.dot_general(
        a, b, (((1,), (1,)), ((), ())),
        precision=jax.lax.Precision.HIGHEST,
        preferred_element_type=jnp.float32)


def _body(facts_ref, fl_ref, agg_ref, rs_ref, w_ref, gamma_ref, beta_ref,
          out_ref, mask_ref, andt_ref, ort_ref):
    f = facts_ref[...]
    fl = fl_ref[...]

    z = fl * (1.0 / FACT_TEMP)
    z = z - jnp.max(z, axis=1, keepdims=True)
    e = jnp.exp(z)
    p = e / jnp.sum(e, axis=1, keepdims=True)
    mask = jnp.minimum(TOP_K_FACTS * p, 1.0)
    mask_ref[...] = mask
    denom = jnp.sum(mask, axis=1, keepdims=True) + 1e-8

    kofnt = _dot_t(mask, f) / denom

    ft = f.T

    def rule(r, carry):
        m_col = mask_ref[pl.ds(r, 1), :].T
        sel = ft * m_col
        and_t = sel + (1.0 - m_col)
        or_t = (1.0 - sel) + 1e-8
        andt_ref[pl.ds(r, 1), :] = _sub_prod(and_t)
        ort_ref[pl.ds(r, 1), :] = _sub_prod(or_t)
        return carry

    jax.lax.fori_loop(0, R, rule, 0, unroll=2)

    aw = agg_ref[...]
    aw = aw - jnp.max(aw, axis=1, keepdims=True)
    ea = jnp.exp(aw)
    w = ea / jnp.sum(ea, axis=1, keepdims=True)
    strength = jax.nn.sigmoid(rs_ref[...])

    mixedt = (andt_ref[...] * w[:, 0:1]
              + (1.0 - ort_ref[...]) * w[:, 1:2]
              + kofnt * w[:, 2:3]) * strength
    act = mixedt.T

    iota = jax.lax.broadcasted_iota(jnp.int32, (B, R), 1)
    removed = jnp.zeros((B, R), jnp.bool_)
    for _ in range(TOP_K_RULES):
        cur = jnp.where(removed, -jnp.inf, act)
        m = jnp.max(cur, axis=1, keepdims=True)
        cand = jnp.where(cur == m, iota, R)
        sel_idx = jnp.min(cand, axis=1, keepdims=True)
        removed = removed | (iota == sel_idx)
    gated = jnp.where(removed, act, 0.0)

    proj = _dot_t(f, w_ref[...])
    pre = proj + gated
    mu = jnp.mean(pre, axis=1, keepdims=True)
    cen = pre - mu
    var = jnp.mean(cen * cen, axis=1, keepdims=True)
    out_ref[...] = cen * jax.lax.rsqrt(var + 1e-5) * gamma_ref[...] \
        + beta_ref[...]


@jax.jit
def kernel(facts, fact_logits, aggregator_logits, rule_strength_raw, W_proj,
           ln_gamma, ln_beta):
    rs = rule_strength_raw.reshape(R, 1)
    gamma = ln_gamma.reshape(1, R)
    beta = ln_beta.reshape(1, R)
    return pl.pallas_call(
        _body,
        out_shape=jax.ShapeDtypeStruct((B, R), jnp.float32),
        scratch_shapes=[
            pltpu.VMEM((R, D), jnp.float32),
            pltpu.VMEM((R, B), jnp.float32),
            pltpu.VMEM((R, B), jnp.float32),
        ],
    )(facts, fact_logits, aggregator_logits, rs, W_proj, gamma, beta)


# speedup vs baseline: 3.6534x; 1.0941x over previous
import jax
import jax.numpy as jnp
from jax.experimental import pallas as pl
from jax.experimental.pallas import tpu as pltpu

B, D, R = 512, 512, 256
TOP_K_FACTS, TOP_K_RULES, FACT_TEMP = 2, 8, 0.7


def _sub_prod(t):
    n = t.shape[0]
    while n > 1:
        h = n // 2
        t = t[:h, :] * t[h:n, :]
        n = h
    return t


def _dot_t(a, b):
    return jax.lax.dot_general(
        a, b, (((1,), (1,)), ((), ())),
        precision=jax.lax.Precision.HIGHEST,
        preferred_element_type=jnp.float32)


def _body(facts_ref, fl_ref, agg_ref, rs_ref, w_ref, gamma_ref, beta_ref,
          out_ref, mask_ref, andt_ref, ort_ref):
    f = facts_ref[...]
    fl = fl_ref[...]

    z = fl * (1.0 / FACT_TEMP)
    z = z - jnp.max(z, axis=1, keepdims=True)
    e = jnp.exp(z)
    p = e / jnp.sum(e, axis=1, keepdims=True)
    mask = jnp.minimum(TOP_K_FACTS * p, 1.0)
    mask_ref[...] = mask
    denom = jnp.sum(mask, axis=1, keepdims=True) + 1e-8

    kofnt = _dot_t(mask, f) / denom

    ft = f.T

    def rule(r, carry):
        m_col = mask_ref[pl.ds(r, 1), :].T
        sel = ft * m_col
        and_t = sel + (1.0 - m_col)
        or_t = (1.0 - sel) + 1e-8
        andt_ref[pl.ds(r, 1), :] = _sub_prod(and_t)
        ort_ref[pl.ds(r, 1), :] = _sub_prod(or_t)
        return carry

    jax.lax.fori_loop(0, R, rule, 0, unroll=4)

    aw = agg_ref[...]
    aw = aw - jnp.max(aw, axis=1, keepdims=True)
    ea = jnp.exp(aw)
    w = ea / jnp.sum(ea, axis=1, keepdims=True)
    strength = jax.nn.sigmoid(rs_ref[...])

    mixedt = (andt_ref[...] * w[:, 0:1]
              + (1.0 - ort_ref[...]) * w[:, 1:2]
              + kofnt * w[:, 2:3]) * strength
    act = mixedt.T

    iota = jax.lax.broadcasted_iota(jnp.int32, (B, R), 1)
    removed = jnp.zeros((B, R), jnp.bool_)
    for _ in range(TOP_K_RULES):
        cur = jnp.where(removed, -jnp.inf, act)
        m = jnp.max(cur, axis=1, keepdims=True)
        cand = jnp.where(cur == m, iota, R)
        sel_idx = jnp.min(cand, axis=1, keepdims=True)
        removed = removed | (iota == sel_idx)
    gated = jnp.where(removed, act, 0.0)

    proj = _dot_t(f, w_ref[...])
    pre = proj + gated
    mu = jnp.mean(pre, axis=1, keepdims=True)
    cen = pre - mu
    var = jnp.mean(cen * cen, axis=1, keepdims=True)
    out_ref[...] = cen * jax.lax.rsqrt(var + 1e-5) * gamma_ref[...] \
        + beta_ref[...]


@jax.jit
def kernel(facts, fact_logits, aggregator_logits, rule_strength_raw, W_proj,
           ln_gamma, ln_beta):
    rs = rule_strength_raw.reshape(R, 1)
    gamma = ln_gamma.reshape(1, R)
    beta = ln_beta.reshape(1, R)
    return pl.pallas_call(
        _body,
        out_shape=jax.ShapeDtypeStruct((B, R), jnp.float32),
        scratch_shapes=[
            pltpu.VMEM((R, D), jnp.float32),
            pltpu.VMEM((R, B), jnp.float32),
            pltpu.VMEM((R, B), jnp.float32),
        ],
    )(facts, fact_logits, aggregator_logits, rs, W_proj, gamma, beta)


# speedup vs baseline: 3.8111x; 1.0431x over previous
import jax
import jax.numpy as jnp
from jax.experimental import pallas as pl
from jax.experimental.pallas import tpu as pltpu

B, D, R = 512, 512, 256
TOP_K_FACTS, TOP_K_RULES, FACT_TEMP = 2, 8, 0.7


def _sub_prod(t):
    n = t.shape[0]
    while n > 1:
        h = n // 2
        t = t[:h, :] * t[h:n, :]
        n = h
    return t


def _dot_t(a, b):
    return jax.lax.dot_general(
        a, b, (((1,), (1,)), ((), ())),
        precision=jax.lax.Precision.HIGHEST,
        preferred_element_type=jnp.float32)


def _body(facts_ref, fl_ref, agg_ref, rs_ref, w_ref, gamma_ref, beta_ref,
          out_ref, mask_ref, andt_ref, ort_ref):
    f = facts_ref[...]
    fl = fl_ref[...]

    z = fl * (1.0 / FACT_TEMP)
    z = z - jnp.max(z, axis=1, keepdims=True)
    e = jnp.exp(z)
    p = e / jnp.sum(e, axis=1, keepdims=True)
    mask = jnp.minimum(TOP_K_FACTS * p, 1.0)
    mask_ref[...] = mask
    denom = jnp.sum(mask, axis=1, keepdims=True) + 1e-8

    kofnt = _dot_t(mask, f) / denom

    ft = f.T

    def rule(r, carry):
        m_col = mask_ref[pl.ds(r, 1), :].T
        sel = ft * m_col
        and_t = sel + (1.0 - m_col)
        or_t = (1.0 - sel) + 1e-8
        andt_ref[pl.ds(r, 1), :] = _sub_prod(and_t)
        ort_ref[pl.ds(r, 1), :] = _sub_prod(or_t)
        return carry

    jax.lax.fori_loop(0, R, rule, 0, unroll=8)

    aw = agg_ref[...]
    aw = aw - jnp.max(aw, axis=1, keepdims=True)
    ea = jnp.exp(aw)
    w = ea / jnp.sum(ea, axis=1, keepdims=True)
    strength = jax.nn.sigmoid(rs_ref[...])

    mixedt = (andt_ref[...] * w[:, 0:1]
              + (1.0 - ort_ref[...]) * w[:, 1:2]
              + kofnt * w[:, 2:3]) * strength
    act = mixedt.T

    iota = jax.lax.broadcasted_iota(jnp.int32, (B, R), 1)
    removed = jnp.zeros((B, R), jnp.bool_)
    for _ in range(TOP_K_RULES):
        cur = jnp.where(removed, -jnp.inf, act)
        m = jnp.max(cur, axis=1, keepdims=True)
        cand = jnp.where(cur == m, iota, R)
        sel_idx = jnp.min(cand, axis=1, keepdims=True)
        removed = removed | (iota == sel_idx)
    gated = jnp.where(removed, act, 0.0)

    proj = _dot_t(f, w_ref[...])
    pre = proj + gated
    mu = jnp.mean(pre, axis=1, keepdims=True)
    cen = pre - mu
    var = jnp.mean(cen * cen, axis=1, keepdims=True)
    out_ref[...] = cen * jax.lax.rsqrt(var + 1e-5) * gamma_ref[...] \
        + beta_ref[...]


@jax.jit
def kernel(facts, fact_logits, aggregator_logits, rule_strength_raw, W_proj,
           ln_gamma, ln_beta):
    rs = rule_strength_raw.reshape(R, 1)
    gamma = ln_gamma.reshape(1, R)
    beta = ln_beta.reshape(1, R)
    return pl.pallas_call(
        _body,
        out_shape=jax.ShapeDtypeStruct((B, R), jnp.float32),
        scratch_shapes=[
            pltpu.VMEM((R, D), jnp.float32),
            pltpu.VMEM((R, B), jnp.float32),
            pltpu.VMEM((R, B), jnp.float32),
        ],
    )(facts, fact_logits, aggregator_logits, rs, W_proj, gamma, beta)
